# baseline (speedup 1.0000x reference)
import functools
import math

import jax
import jax.numpy as jnp
from jax import lax
from jax.experimental import pallas as pl
from jax.experimental.pallas import tpu as pltpu
from jax.experimental.pallas import tpu_sc as plsc

_EMBED = 64
_HALF = 32
_BATCH = 16384
_HALFB = _BATCH // 2
_NSTATES = 3
_LUT_ROWS = 32
_PLUT_ROWS = 736

_NC = 2
_NS = 16
_NW = _NC * _NS
_PPW = _HALFB // _NW


def _lut_body(st_ref, sub_ref, w1_ref, b1_ref, w2_ref, b2_ref, plut_ref):
    t = lax.broadcasted_iota(jnp.int32, (_LUT_ROWS, 1), 0)
    a = t // 9
    b = (t // 3) % 3
    c = t % 3

    st = st_ref[...]
    sub = sub_ref[...]

    pooled_state = jnp.zeros((_LUT_ROWS, _HALF), jnp.float32)
    for k in range(_NSTATES):
        cnt = ((a == k).astype(jnp.float32)
               + (b == k).astype(jnp.float32)
               + (c == k).astype(jnp.float32))
        pooled_state = pooled_state + cnt * st[k : k + 1, :]
    pooled_state = pooled_state / 3.0

    sub_mean = (sub[0:1, :] + sub[1:2, :] + sub[2:3, :]) / 3.0
    pooled = jnp.concatenate(
        [pooled_state, jnp.broadcast_to(sub_mean, (_LUT_ROWS, _HALF))], axis=1
    )

    mean_idx = (a + b + c).astype(jnp.float32) / 3.0
    phase = (2.0 * math.pi / _NSTATES) * mean_idx
    p = lax.broadcasted_iota(jnp.int32, (1, _EMBED), 1)
    even_base = (p - p % 2).astype(jnp.float32)
    freq = jnp.exp(even_base * (-(math.log(10000.0) / _EMBED)))
    angles = phase * freq
    pe = jnp.where(p % 2 == 0, jnp.sin(angles), jnp.cos(angles))

    x = jnp.concatenate([pooled, pe], axis=1)

    h = jnp.dot(x, w1_ref[...], preferred_element_type=jnp.float32) + b1_ref[...]
    h = 0.5 * h * (1.0 + lax.erf(h / math.sqrt(2.0)))
    out = jnp.dot(h, w2_ref[...], preferred_element_type=jnp.float32) + b2_ref[...]

    norm = jnp.sqrt(jnp.sum(out * out, axis=1, keepdims=True))
    factor = jnp.minimum(jnp.ones_like(norm), 0.95 / (norm + 1e-8))
    lut = out * factor

    tp = lax.broadcasted_iota(jnp.int32, (_PLUT_ROWS, 1), 0)
    k = lax.broadcasted_iota(jnp.int32, (1, _LUT_ROWS), 1)
    oh_hi = (tp // 27 == k).astype(jnp.float32)
    oh_lo = (tp % 27 == k).astype(jnp.float32)
    left = jnp.dot(oh_hi, lut, preferred_element_type=jnp.float32)
    right = jnp.dot(oh_lo, lut, preferred_element_type=jnp.float32)
    plut_ref[...] = jnp.concatenate([left, right], axis=1)


def _sc_gather_body(
    plut_hbm, idx_hbm, out_hbm, tri_v, keys_v, rows_v, sem, gsem0, gsem1
):
    wid = lax.axis_index("s") * _NC + lax.axis_index("c")
    base = wid * _PPW

    idx_copies = []
    for col in range(3):
        for h in range(2):
            idx_copies.append(
                pltpu.async_copy(
                    idx_hbm.at[pl.ds(col * _BATCH + h * _HALFB + base, _PPW)],
                    tri_v.at[pl.ds((col * 2 + h) * _PPW, _PPW)],
                    sem,
                )
            )
    for c in idx_copies:
        c.wait()

    def grp(j, carry):
        lo = j * 16
        a0 = tri_v[pl.ds(lo, 16)]
        a1 = tri_v[pl.ds(_PPW + lo, 16)]
        b0 = tri_v[pl.ds(2 * _PPW + lo, 16)]
        b1 = tri_v[pl.ds(3 * _PPW + lo, 16)]
        c0 = tri_v[pl.ds(4 * _PPW + lo, 16)]
        c1 = tri_v[pl.ds(5 * _PPW + lo, 16)]
        keys_v[pl.ds(lo, 16)] = (
            (a0 * 9 + b0 * 3 + c0) * 27 + a1 * 9 + b1 * 3 + c1
        )
        return carry

    half_p = _PPW // 2
    lax.fori_loop(0, _PPW // 32, grp, 0)
    g0 = pltpu.async_copy(
        plut_hbm.at[keys_v.at[pl.ds(0, half_p)]],
        rows_v.at[pl.ds(0, half_p)],
        gsem0,
    )
    lax.fori_loop(_PPW // 32, _PPW // 16, grp, 0)
    g1 = pltpu.async_copy(
        plut_hbm.at[keys_v.at[pl.ds(half_p, half_p)]],
        rows_v.at[pl.ds(half_p, half_p)],
        gsem1,
    )
    g0.wait()
    w0 = pltpu.async_copy(
        rows_v.at[pl.ds(0, half_p)], out_hbm.at[pl.ds(base, half_p)], sem
    )
    g1.wait()
    w1 = pltpu.async_copy(
        rows_v.at[pl.ds(half_p, half_p)],
        out_hbm.at[pl.ds(base + half_p, half_p)],
        gsem0,
    )
    w0.wait()
    w1.wait()


@functools.lru_cache(maxsize=1)
def _make_sc_gather():
    return pl.kernel(
        _sc_gather_body,
        out_type=jax.ShapeDtypeStruct((_HALFB, 2 * _EMBED), jnp.float32),
        scratch_types=[
            pltpu.VMEM((_PPW * 6,), jnp.int32),
            pltpu.VMEM((_PPW,), jnp.int32),
            pltpu.VMEM((_PPW, 2 * _EMBED), jnp.float32),
            pltpu.SemaphoreType.DMA,
            pltpu.SemaphoreType.DMA,
            pltpu.SemaphoreType.DMA,
        ],
        mesh=plsc.VectorSubcoreMesh(core_axis_name="c", subcore_axis_name="s"),
        compiler_params=pltpu.CompilerParams(use_tc_tiling_on_sc=False),
    )


def _repack_body(pairs_ref, out_ref):
    h = pl.program_id(1)
    blk = pairs_ref[...]
    half_rows = blk.shape[0] // 2
    r = lax.broadcasted_iota(jnp.int32, (128, 1), 0)
    c = lax.broadcasted_iota(jnp.int32, (1, 128), 1)
    ident = (r == c).astype(jnp.float32)

    def half(lane0):
        out_ref[:, :half_rows] = blk[:half_rows, lane0 : lane0 + _EMBED].T
        for i in range(half_rows // 128, blk.shape[0] // 128):
            chunk = blk[i * 128 : (i + 1) * 128, lane0 : lane0 + _EMBED]
            out_ref[:, i * 128 : (i + 1) * 128] = lax.dot_general(
                chunk,
                ident,
                (((0,), (0,)), ((), ())),
                preferred_element_type=jnp.float32,
            )

    @pl.when(h == 0)
    def _left():
        half(0)

    @pl.when(h == 1)
    def _right():
        half(_EMBED)


def _repack(pairs):
    nblk = 8
    blk = _HALFB // nblk
    return pl.pallas_call(
        _repack_body,
        grid=(nblk, 2),
        in_specs=[pl.BlockSpec((blk, 2 * _EMBED), lambda k, h: (k, 0))],
        out_specs=pl.BlockSpec((_EMBED, blk), lambda k, h: (0, h * nblk + k)),
        out_shape=jax.ShapeDtypeStruct((_EMBED, _BATCH), jnp.float32),
    )(pairs)


def kernel(state_indices, state_table, subunit_table, W1, b1, W2, b2):
    plut = pl.pallas_call(
        _lut_body,
        out_shape=jax.ShapeDtypeStruct((_PLUT_ROWS, 2 * _EMBED), jnp.float32),
    )(state_table, subunit_table, W1, b1, W2, b2)
    flat_idx = state_indices.T.reshape(-1).astype(jnp.int32)
    pairs = _make_sc_gather()(plut, flat_idx)
    return _repack(pairs).T


# speedup vs baseline: 1.1124x; 1.1124x over previous
import functools
import math

import jax
import jax.numpy as jnp
from jax import lax
from jax.experimental import pallas as pl
from jax.experimental.pallas import tpu as pltpu
from jax.experimental.pallas import tpu_sc as plsc

_EMBED = 64
_HALF = 32
_BATCH = 16384
_HALFB = _BATCH // 2
_NSTATES = 3
_LUT_ROWS = 32
_PLUT_ROWS = 736

_NC = 2
_NS = 16
_NW = _NC * _NS
_PPW = _HALFB // _NW


def _lut_body(st_ref, sub_ref, w1_ref, b1_ref, w2_ref, b2_ref, plut_ref):
    t = lax.broadcasted_iota(jnp.int32, (_LUT_ROWS, 1), 0)
    a = t // 9
    b = (t // 3) % 3
    c = t % 3

    st = st_ref[...]
    sub = sub_ref[...]

    pooled_state = jnp.zeros((_LUT_ROWS, _HALF), jnp.float32)
    for k in range(_NSTATES):
        cnt = ((a == k).astype(jnp.float32)
               + (b == k).astype(jnp.float32)
               + (c == k).astype(jnp.float32))
        pooled_state = pooled_state + cnt * st[k : k + 1, :]
    pooled_state = pooled_state / 3.0

    sub_mean = (sub[0:1, :] + sub[1:2, :] + sub[2:3, :]) / 3.0
    pooled = jnp.concatenate(
        [pooled_state, jnp.broadcast_to(sub_mean, (_LUT_ROWS, _HALF))], axis=1
    )

    mean_idx = (a + b + c).astype(jnp.float32) / 3.0
    phase = (2.0 * math.pi / _NSTATES) * mean_idx
    p = lax.broadcasted_iota(jnp.int32, (1, _EMBED), 1)
    even_base = (p - p % 2).astype(jnp.float32)
    freq = jnp.exp(even_base * (-(math.log(10000.0) / _EMBED)))
    angles = phase * freq
    pe = jnp.where(p % 2 == 0, jnp.sin(angles), jnp.cos(angles))

    x = jnp.concatenate([pooled, pe], axis=1)

    h = jnp.dot(x, w1_ref[...], preferred_element_type=jnp.float32) + b1_ref[...]
    h = 0.5 * h * (1.0 + lax.erf(h / math.sqrt(2.0)))
    out = jnp.dot(h, w2_ref[...], preferred_element_type=jnp.float32) + b2_ref[...]

    norm = jnp.sqrt(jnp.sum(out * out, axis=1, keepdims=True))
    factor = jnp.minimum(jnp.ones_like(norm), 0.95 / (norm + 1e-8))
    lut = out * factor

    tp = lax.broadcasted_iota(jnp.int32, (_PLUT_ROWS, 1), 0)
    k = lax.broadcasted_iota(jnp.int32, (1, _LUT_ROWS), 1)
    oh_hi = (tp // 27 == k).astype(jnp.float32)
    oh_lo = (tp % 27 == k).astype(jnp.float32)
    left = jnp.dot(oh_hi, lut, preferred_element_type=jnp.float32)
    right = jnp.dot(oh_lo, lut, preferred_element_type=jnp.float32)
    plut_ref[...] = jnp.concatenate([left, right], axis=1)


def _sc_gather_body(
    plut_hbm, idx_hbm, out_hbm, tri_v, keys_v, rows_v, sem, gsem0, gsem1
):
    wid = lax.axis_index("s") * _NC + lax.axis_index("c")
    base = wid * _PPW

    idx_copies = []
    for col in range(3):
        for h in range(2):
            idx_copies.append(
                pltpu.async_copy(
                    idx_hbm.at[pl.ds(col * _BATCH + h * _HALFB + base, _PPW)],
                    tri_v.at[pl.ds((col * 2 + h) * _PPW, _PPW)],
                    sem,
                )
            )
    for c in idx_copies:
        c.wait()

    def grp(j, carry):
        lo = j * 16
        a0 = tri_v[pl.ds(lo, 16)]
        a1 = tri_v[pl.ds(_PPW + lo, 16)]
        b0 = tri_v[pl.ds(2 * _PPW + lo, 16)]
        b1 = tri_v[pl.ds(3 * _PPW + lo, 16)]
        c0 = tri_v[pl.ds(4 * _PPW + lo, 16)]
        c1 = tri_v[pl.ds(5 * _PPW + lo, 16)]
        keys_v[pl.ds(lo, 16)] = (
            (a0 * 9 + b0 * 3 + c0) * 27 + a1 * 9 + b1 * 3 + c1
        )
        return carry

    half_p = _PPW // 2
    lax.fori_loop(0, _PPW // 32, grp, 0)
    g0 = pltpu.async_copy(
        plut_hbm.at[keys_v.at[pl.ds(0, half_p)]],
        rows_v.at[pl.ds(0, half_p)],
        gsem0,
    )
    lax.fori_loop(_PPW // 32, _PPW // 16, grp, 0)
    g1 = pltpu.async_copy(
        plut_hbm.at[keys_v.at[pl.ds(half_p, half_p)]],
        rows_v.at[pl.ds(half_p, half_p)],
        gsem1,
    )
    g0.wait()
    w0 = pltpu.async_copy(
        rows_v.at[pl.ds(0, half_p)], out_hbm.at[pl.ds(base, half_p)], sem
    )
    g1.wait()
    w1 = pltpu.async_copy(
        rows_v.at[pl.ds(half_p, half_p)],
        out_hbm.at[pl.ds(base + half_p, half_p)],
        gsem0,
    )
    w0.wait()
    w1.wait()


@functools.lru_cache(maxsize=1)
def _make_sc_gather():
    return pl.kernel(
        _sc_gather_body,
        out_type=jax.ShapeDtypeStruct((_HALFB, 2 * _EMBED), jnp.float32),
        scratch_types=[
            pltpu.VMEM((_PPW * 6,), jnp.int32),
            pltpu.VMEM((_PPW,), jnp.int32),
            pltpu.VMEM((_PPW, 2 * _EMBED), jnp.float32),
            pltpu.SemaphoreType.DMA,
            pltpu.SemaphoreType.DMA,
            pltpu.SemaphoreType.DMA,
        ],
        mesh=plsc.VectorSubcoreMesh(core_axis_name="c", subcore_axis_name="s"),
        compiler_params=pltpu.CompilerParams(use_tc_tiling_on_sc=False),
    )


def _repack_body(pairs_ref, out_ref):
    h = pl.program_id(1)

    @pl.when(h == 0)
    def _left():
        out_ref[...] = pairs_ref[...][:, : _EMBED].T

    @pl.when(h == 1)
    def _right():
        out_ref[...] = pairs_ref[...][:, _EMBED :].T


def _repack(pairs):
    nblk = 4
    blk = _HALFB // nblk
    return pl.pallas_call(
        _repack_body,
        grid=(nblk, 2),
        in_specs=[pl.BlockSpec((blk, 2 * _EMBED), lambda k, h: (k, 0))],
        out_specs=pl.BlockSpec((_EMBED, blk), lambda k, h: (0, h * nblk + k)),
        out_shape=jax.ShapeDtypeStruct((_EMBED, _BATCH), jnp.float32),
    )(pairs)


def kernel(state_indices, state_table, subunit_table, W1, b1, W2, b2):
    plut = pl.pallas_call(
        _lut_body,
        out_shape=jax.ShapeDtypeStruct((_PLUT_ROWS, 2 * _EMBED), jnp.float32),
    )(state_table, subunit_table, W1, b1, W2, b2)
    flat_idx = state_indices.T.reshape(-1).astype(jnp.int32)
    pairs = _make_sc_gather()(plut, flat_idx)
    return _repack(pairs).T


# speedup vs baseline: 1.1500x; 1.0338x over previous
import functools
import math

import jax
import jax.numpy as jnp
from jax import lax
from jax.experimental import pallas as pl
from jax.experimental.pallas import tpu as pltpu
from jax.experimental.pallas import tpu_sc as plsc

_EMBED = 64
_HALF = 32
_BATCH = 16384
_HALFB = _BATCH // 2
_NSTATES = 3
_LUT_ROWS = 32
_PLUT_ROWS = 736

_NC = 2
_NS = 16
_NW = _NC * _NS
_PPW = _HALFB // _NW


def _lut_body(st_ref, sub_ref, w1_ref, b1_ref, w2_ref, b2_ref, plut_ref):
    t = lax.broadcasted_iota(jnp.int32, (_LUT_ROWS, 1), 0)
    a = t // 9
    b = (t // 3) % 3
    c = t % 3

    st = st_ref[...]
    sub = sub_ref[...]

    pooled_state = jnp.zeros((_LUT_ROWS, _HALF), jnp.float32)
    for k in range(_NSTATES):
        cnt = ((a == k).astype(jnp.float32)
               + (b == k).astype(jnp.float32)
               + (c == k).astype(jnp.float32))
        pooled_state = pooled_state + cnt * st[k : k + 1, :]
    pooled_state = pooled_state / 3.0

    sub_mean = (sub[0:1, :] + sub[1:2, :] + sub[2:3, :]) / 3.0
    pooled = jnp.concatenate(
        [pooled_state, jnp.broadcast_to(sub_mean, (_LUT_ROWS, _HALF))], axis=1
    )

    mean_idx = (a + b + c).astype(jnp.float32) / 3.0
    phase = (2.0 * math.pi / _NSTATES) * mean_idx
    p = lax.broadcasted_iota(jnp.int32, (1, _EMBED), 1)
    even_base = (p - p % 2).astype(jnp.float32)
    freq = jnp.exp(even_base * (-(math.log(10000.0) / _EMBED)))
    angles = phase * freq
    pe = jnp.where(p % 2 == 0, jnp.sin(angles), jnp.cos(angles))

    x = jnp.concatenate([pooled, pe], axis=1)

    h = jnp.dot(x, w1_ref[...], preferred_element_type=jnp.float32) + b1_ref[...]
    h = 0.5 * h * (1.0 + lax.erf(h / math.sqrt(2.0)))
    out = jnp.dot(h, w2_ref[...], preferred_element_type=jnp.float32) + b2_ref[...]

    norm = jnp.sqrt(jnp.sum(out * out, axis=1, keepdims=True))
    factor = jnp.minimum(jnp.ones_like(norm), 0.95 / (norm + 1e-8))
    lut = out * factor

    tp = lax.broadcasted_iota(jnp.int32, (_PLUT_ROWS, 1), 0)
    k = lax.broadcasted_iota(jnp.int32, (1, _LUT_ROWS), 1)
    oh_hi = (tp // 27 == k).astype(jnp.float32)
    oh_lo = (tp % 27 == k).astype(jnp.float32)
    left = jnp.dot(oh_hi, lut, preferred_element_type=jnp.float32)
    right = jnp.dot(oh_lo, lut, preferred_element_type=jnp.float32)
    plut_ref[...] = jnp.concatenate([left, right], axis=1)


def _sc_gather_body(
    plut_hbm, idx_hbm, out_hbm, tri_v, keys_v, rows_v, sem, gsem0, gsem1
):
    wid = lax.axis_index("s") * _NC + lax.axis_index("c")
    base = wid * _PPW

    idx_copies = []
    for col in range(3):
        for h in range(2):
            idx_copies.append(
                pltpu.async_copy(
                    idx_hbm.at[pl.ds(col * _BATCH + h * _HALFB + base, _PPW)],
                    tri_v.at[pl.ds((col * 2 + h) * _PPW, _PPW)],
                    sem,
                )
            )
    for c in idx_copies:
        c.wait()

    def grp(j, carry):
        lo = j * 16
        a0 = tri_v[pl.ds(lo, 16)]
        a1 = tri_v[pl.ds(_PPW + lo, 16)]
        b0 = tri_v[pl.ds(2 * _PPW + lo, 16)]
        b1 = tri_v[pl.ds(3 * _PPW + lo, 16)]
        c0 = tri_v[pl.ds(4 * _PPW + lo, 16)]
        c1 = tri_v[pl.ds(5 * _PPW + lo, 16)]
        keys_v[pl.ds(lo, 16)] = (
            (a0 * 9 + b0 * 3 + c0) * 27 + a1 * 9 + b1 * 3 + c1
        )
        return carry

    half_p = _PPW // 2
    lax.fori_loop(0, _PPW // 32, grp, 0)
    g0 = pltpu.async_copy(
        plut_hbm.at[keys_v.at[pl.ds(0, half_p)]],
        rows_v.at[pl.ds(0, half_p)],
        gsem0,
    )
    lax.fori_loop(_PPW // 32, _PPW // 16, grp, 0)
    g1 = pltpu.async_copy(
        plut_hbm.at[keys_v.at[pl.ds(half_p, half_p)]],
        rows_v.at[pl.ds(half_p, half_p)],
        gsem1,
    )
    g0.wait()
    w0 = pltpu.async_copy(
        rows_v.at[pl.ds(0, half_p)], out_hbm.at[pl.ds(base, half_p)], sem
    )
    g1.wait()
    w1 = pltpu.async_copy(
        rows_v.at[pl.ds(half_p, half_p)],
        out_hbm.at[pl.ds(base + half_p, half_p)],
        gsem0,
    )
    w0.wait()
    w1.wait()


@functools.lru_cache(maxsize=1)
def _make_sc_gather():
    return pl.kernel(
        _sc_gather_body,
        out_type=jax.ShapeDtypeStruct((_HALFB, 2 * _EMBED), jnp.float32),
        scratch_types=[
            pltpu.VMEM((_PPW * 6,), jnp.int32),
            pltpu.VMEM((_PPW,), jnp.int32),
            pltpu.VMEM((_PPW, 2 * _EMBED), jnp.float32),
            pltpu.SemaphoreType.DMA,
            pltpu.SemaphoreType.DMA,
            pltpu.SemaphoreType.DMA,
        ],
        mesh=plsc.VectorSubcoreMesh(core_axis_name="c", subcore_axis_name="s"),
        compiler_params=pltpu.CompilerParams(use_tc_tiling_on_sc=False),
    )


def _repack_body(pairs_ref, out_ref):
    h = pl.program_id(1)

    @pl.when(h == 0)
    def _left():
        out_ref[...] = pairs_ref[...][:, : _EMBED].T

    @pl.when(h == 1)
    def _right():
        out_ref[...] = pairs_ref[...][:, _EMBED :].T


def _repack(pairs):
    nblk = 2
    blk = _HALFB // nblk
    return pl.pallas_call(
        _repack_body,
        grid=(nblk, 2),
        in_specs=[pl.BlockSpec((blk, 2 * _EMBED), lambda k, h: (k, 0))],
        out_specs=pl.BlockSpec((_EMBED, blk), lambda k, h: (0, h * nblk + k)),
        out_shape=jax.ShapeDtypeStruct((_EMBED, _BATCH), jnp.float32),
    )(pairs)


def kernel(state_indices, state_table, subunit_table, W1, b1, W2, b2):
    plut = pl.pallas_call(
        _lut_body,
        out_shape=jax.ShapeDtypeStruct((_PLUT_ROWS, 2 * _EMBED), jnp.float32),
    )(state_table, subunit_table, W1, b1, W2, b2)
    flat_idx = state_indices.T.reshape(-1).astype(jnp.int32)
    pairs = _make_sc_gather()(plut, flat_idx)
    return _repack(pairs).T
